# baseline (speedup 1.0000x reference)
import functools

import jax
import jax.numpy as jnp
from jax import lax
from jax.experimental import pallas as pl
from jax.experimental.pallas import tpu as pltpu
from jax.experimental.pallas import tpu_sc as plsc

N = 10000
E = 320000
D = 128
G = 64

NC = 2
NS = 16
NW = NC * NS
KB = 128
NBLK = -((E + NW * KB - 1) // (NW * KB)) // 8 * -8
EPAD = NW * KB * NBLK
ACC_N = 10112
RZ = ACC_N // NS

DEGW = 128
NBUF = 2
NPH = 2
NBLK0 = 32
NBLK1 = 2 * NBLK - NBLK0
HBMAX = max(NBLK0, NBLK1) // NPH
BLK = 1000
NCHUNK = N // BLK


@functools.cache
def _sc_kernels():
    mesh = plsc.VectorSubcoreMesh(core_axis_name="c", subcore_axis_name="s")

    @functools.partial(
        pl.kernel,
        out_type=jax.ShapeDtypeStruct((NC, ACC_N, DEGW), jnp.float32),
        mesh=mesh,
        scratch_types=[
            pltpu.VMEM((NBLK, KB), jnp.int32),
            pltpu.VMEM((KB, DEGW), jnp.float32),
            pltpu.VMEM_SHARED((ACC_N, DEGW), jnp.float32),
        ],
    )
    def sc_degree(dst_hbm, ones_hbm, zeros_hbm, out_hbm, dstb, onesb, acc):
        c = lax.axis_index("c")
        s = lax.axis_index("s")
        w = s * NC + c
        pltpu.sync_copy(zeros_hbm.at[pl.ds(s * RZ, RZ)],
                        acc.at[pl.ds(s * RZ, RZ)])
        pltpu.sync_copy(ones_hbm, onesb)
        pltpu.sync_copy(dst_hbm.at[pl.ds(w * NBLK, NBLK)], dstb)
        plsc.subcore_barrier()

        def body(g, carry):
            pltpu.sync_copy(onesb, acc.at[dstb.at[g]], add=True)
            return carry

        lax.fori_loop(0, NBLK, body, 0)
        plsc.subcore_barrier()
        pltpu.sync_copy(acc.at[pl.ds(s * RZ, RZ)],
                        out_hbm.at[c, pl.ds(s * RZ, RZ)])

    @functools.partial(
        pl.kernel,
        out_type=jax.ShapeDtypeStruct((NC, ACC_N, D), jnp.float32),
        mesh=mesh,
        scratch_types=[
            pltpu.VMEM((HBMAX, KB), jnp.int32),
            pltpu.VMEM((HBMAX, KB), jnp.int32),
            [pltpu.VMEM((KB, D), jnp.float32)] * NBUF,
            pltpu.VMEM_SHARED((ACC_N, D), jnp.float32),
            [pltpu.SemaphoreType.DMA] * NBUF,
            [pltpu.SemaphoreType.DMA] * NBUF,
        ],
    )
    def sc_spmm(p_hbm, src_hbm, dst_hbm, zeros_hbm, out_hbm,
                srcb, dstb, rows, acc, gsem, ssem):
        c = lax.axis_index("c")
        s = lax.axis_index("s")
        pltpu.sync_copy(zeros_hbm.at[pl.ds(s * RZ, RZ)],
                        acc.at[pl.ds(s * RZ, RZ)])
        plsc.subcore_barrier()

        def run_edges(first_blk, nblk_c):
            hb = nblk_c // NPH
            for ph in range(NPH):
                base = first_blk + ph * hb
                pltpu.sync_copy(src_hbm.at[pl.ds(base, hb)],
                                srcb.at[pl.ds(0, hb)])
                pltpu.sync_copy(dst_hbm.at[pl.ds(base, hb)],
                                dstb.at[pl.ds(0, hb)])
                pltpu.async_copy(p_hbm.at[srcb.at[0]], rows[0], gsem[0])

                def body(i, carry):
                    for j in range(NBUF):
                        g = i * NBUF + j
                        bn = 1 - j
                        pltpu.make_async_copy(
                            p_hbm.at[srcb.at[g]], rows[j], gsem[j]).wait()
                        pltpu.async_copy(rows[j], acc.at[dstb.at[g]],
                                         ssem[j], add=True)
                        if j == 0:
                            @pl.when(i > 0)
                            def _():
                                pltpu.make_async_copy(
                                    rows[bn], acc.at[dstb.at[0]],
                                    ssem[bn]).wait()

                            pltpu.async_copy(
                                p_hbm.at[srcb.at[g + 1]], rows[bn], gsem[bn])
                        else:
                            @pl.when(i < hb // NBUF - 1)
                            def _():
                                pltpu.make_async_copy(
                                    rows[bn], acc.at[dstb.at[0]],
                                    ssem[bn]).wait()
                                pltpu.async_copy(
                                    p_hbm.at[srcb.at[g + 1]], rows[bn],
                                    gsem[bn])

                    return carry

                lax.fori_loop(0, hb // NBUF, body, 0)
                for b in range(NBUF):
                    pltpu.make_async_copy(
                        rows[b], acc.at[dstb.at[0]], ssem[b]).wait()

        @pl.when(c == 0)
        def _():
            run_edges(s * NBLK0, NBLK0)

        @pl.when(c == 1)
        def _():
            run_edges(NS * NBLK0 + s * NBLK1, NBLK1)

        plsc.subcore_barrier()
        pltpu.sync_copy(acc.at[pl.ds(s * RZ, RZ)],
                        out_hbm.at[c, pl.ds(s * RZ, RZ)])

    return sc_degree, sc_spmm


def _tc_entry(degA, degB, x, W):
    def body(dA, dB, x_ref, W_ref, out_ref):
        dis = lax.rsqrt(dA[...] + dB[...] + 1.0)
        out_ref[...] = jnp.dot(x_ref[...], W_ref[...],
                               preferred_element_type=jnp.float32) * dis

    return pl.pallas_call(
        body,
        grid=(NCHUNK,),
        in_specs=[
            pl.BlockSpec((BLK, 1), lambda i: (i, 0)),
            pl.BlockSpec((BLK, 1), lambda i: (i, 0)),
            pl.BlockSpec((BLK, D), lambda i: (i, 0)),
            pl.BlockSpec((D, D), lambda i: (0, 0)),
        ],
        out_specs=pl.BlockSpec((BLK, D), lambda i: (i, 0)),
        out_shape=jax.ShapeDtypeStruct((N, D), jnp.float32),
    )(degA, degB, x, W)


def _tc_mid(degA, degB, sA, sB, p, b, W):
    def body(dA, dB, sA_ref, sB_ref, p_ref, b_ref, W_ref, out_ref):
        dis = lax.rsqrt(dA[...] + dB[...] + 1.0)
        h = jnp.maximum(
            dis * (sA_ref[...] + sB_ref[...] + p_ref[...]) + b_ref[...], 0.0)
        out_ref[...] = jnp.dot(h, W_ref[...],
                               preferred_element_type=jnp.float32) * dis

    return pl.pallas_call(
        body,
        grid=(NCHUNK,),
        in_specs=[
            pl.BlockSpec((BLK, 1), lambda i: (i, 0)),
            pl.BlockSpec((BLK, 1), lambda i: (i, 0)),
            pl.BlockSpec((BLK, D), lambda i: (i, 0)),
            pl.BlockSpec((BLK, D), lambda i: (i, 0)),
            pl.BlockSpec((BLK, D), lambda i: (i, 0)),
            pl.BlockSpec((1, D), lambda i: (0, 0)),
            pl.BlockSpec((D, D), lambda i: (0, 0)),
        ],
        out_specs=pl.BlockSpec((BLK, D), lambda i: (i, 0)),
        out_shape=jax.ShapeDtypeStruct((N, D), jnp.float32),
    )(degA, degB, sA, sB, p, b, W)


def _tc_final(degA, degB, sA, sB, p, b, batch3, fW1, fb1, fW2, fb2):
    def body(dA, dB, sA_ref, sB_ref, p_ref, b_ref, bat_ref,
             fW1_ref, fb1_ref, fW2_ref, fb2_ref, out_ref, pool_acc, cnt_acc):
        i = pl.program_id(0)

        @pl.when(i == 0)
        def _():
            pool_acc[...] = jnp.zeros_like(pool_acc)
            cnt_acc[...] = jnp.zeros_like(cnt_acc)

        dis = lax.rsqrt(dA[...] + dB[...] + 1.0)
        h = jnp.maximum(
            dis * (sA_ref[...] + sB_ref[...] + p_ref[...]) + b_ref[...], 0.0)
        bb = bat_ref[0, 0, :]
        oh = (bb[None, :] == lax.broadcasted_iota(jnp.int32, (G, BLK), 0)
              ).astype(jnp.float32)
        pool_acc[...] += jnp.dot(oh, h, preferred_element_type=jnp.float32)
        cnt_acc[...] += jnp.sum(oh, axis=1, keepdims=True)

        @pl.when(i == NCHUNK - 1)
        def _():
            pooled = pool_acc[...] / jnp.maximum(cnt_acc[...], 1.0)
            z = jnp.maximum(
                jnp.dot(pooled, fW1_ref[...],
                        preferred_element_type=jnp.float32) + fb1_ref[...], 0.0)
            out_ref[...] = jnp.dot(z, fW2_ref[...],
                                   preferred_element_type=jnp.float32) + fb2_ref[...]

    return pl.pallas_call(
        body,
        grid=(NCHUNK,),
        in_specs=[
            pl.BlockSpec((BLK, 1), lambda i: (i, 0)),
            pl.BlockSpec((BLK, 1), lambda i: (i, 0)),
            pl.BlockSpec((BLK, D), lambda i: (i, 0)),
            pl.BlockSpec((BLK, D), lambda i: (i, 0)),
            pl.BlockSpec((BLK, D), lambda i: (i, 0)),
            pl.BlockSpec((1, D), lambda i: (0, 0)),
            pl.BlockSpec((1, 1, BLK), lambda i: (i, 0, 0)),
            pl.BlockSpec((D, G), lambda i: (0, 0)),
            pl.BlockSpec((1, G), lambda i: (0, 0)),
            pl.BlockSpec((G, 1), lambda i: (0, 0)),
            pl.BlockSpec((1, 1), lambda i: (0, 0)),
        ],
        out_specs=pl.BlockSpec((G, 1), lambda i: (0, 0)),
        out_shape=jax.ShapeDtypeStruct((G, 1), jnp.float32),
        scratch_shapes=[
            pltpu.VMEM((G, D), jnp.float32),
            pltpu.VMEM((G, 1), jnp.float32),
        ],
    )(degA, degB, sA, sB, p, b, batch3, fW1, fb1, fW2, fb2)


def kernel(x, edge_index, batch, W1, b1, W2, b2, W3, b3, fW1, fb1, fW2, fb2):
    src = edge_index[0]
    dst = edge_index[1]
    pad = EPAD - E
    src2 = jnp.concatenate([src, jnp.zeros((pad,), jnp.int32)]).reshape(-1, KB)
    dst2 = jnp.concatenate([dst, jnp.full((pad,), N, jnp.int32)]).reshape(-1, KB)

    ones_col = jnp.ones((KB, DEGW), jnp.float32)
    zeros_col = jnp.zeros((ACC_N, DEGW), jnp.float32)
    zeros_tab = jnp.zeros((ACC_N, D), jnp.float32)

    sc_degree, sc_spmm = _sc_kernels()
    deg = sc_degree(dst2, ones_col, zeros_col)
    degA, degB = deg[0, :, 0:1], deg[1, :, 0:1]

    b1r = b1.reshape(1, D)
    b2r = b2.reshape(1, D)
    b3r = b3.reshape(1, D)
    batch3 = batch.reshape(NCHUNK, 1, BLK)

    p1 = _tc_entry(degA, degB, x, W1)
    s1 = sc_spmm(p1, src2, dst2, zeros_tab)
    p2 = _tc_mid(degA, degB, s1[0], s1[1], p1, b1r, W2)
    s2 = sc_spmm(p2, src2, dst2, zeros_tab)
    p3 = _tc_mid(degA, degB, s2[0], s2[1], p2, b2r, W3)
    s3 = sc_spmm(p3, src2, dst2, zeros_tab)
    out = _tc_final(degA, degB, s3[0], s3[1], p3, b3r, batch3,
                    fW1, fb1.reshape(1, G), fW2, fb2.reshape(1, 1))
    return out.reshape(-1)


# speedup vs baseline: 1.2169x; 1.2169x over previous
import functools

import jax
import jax.numpy as jnp
from jax import lax
from jax.experimental import pallas as pl
from jax.experimental.pallas import tpu as pltpu
from jax.experimental.pallas import tpu_sc as plsc

N = 10000
E = 320000
D = 128
G = 64

NC = 2
NS = 16
NW = NC * NS
KB = 128
NBLK = -((E + NW * KB - 1) // (NW * KB)) // 8 * -8
EPAD = NW * KB * NBLK
ACC_N = 10112
RZ = ACC_N // NS

DEGW = 128
NBUF = 2
NPH = 2
NBLK0 = 128
NBLK1 = 2 * NBLK - NBLK0
HBMAX = max(NBLK0, NBLK1) // NPH
BLK = 1000
NCHUNK = N // BLK


@functools.cache
def _sc_kernels():
    mesh = plsc.VectorSubcoreMesh(core_axis_name="c", subcore_axis_name="s")

    @functools.partial(
        pl.kernel,
        out_type=jax.ShapeDtypeStruct((NC, ACC_N, DEGW), jnp.float32),
        mesh=mesh,
        scratch_types=[
            pltpu.VMEM((NBLK, KB), jnp.int32),
            pltpu.VMEM((KB, DEGW), jnp.float32),
            pltpu.VMEM_SHARED((ACC_N, DEGW), jnp.float32),
        ],
    )
    def sc_degree(dst_hbm, ones_hbm, zeros_hbm, out_hbm, dstb, onesb, acc):
        c = lax.axis_index("c")
        s = lax.axis_index("s")
        w = s * NC + c
        pltpu.sync_copy(zeros_hbm.at[pl.ds(s * RZ, RZ)],
                        acc.at[pl.ds(s * RZ, RZ)])
        pltpu.sync_copy(ones_hbm, onesb)
        pltpu.sync_copy(dst_hbm.at[pl.ds(w * NBLK, NBLK)], dstb)
        plsc.subcore_barrier()

        def body(g, carry):
            pltpu.sync_copy(onesb, acc.at[dstb.at[g]], add=True)
            return carry

        lax.fori_loop(0, NBLK, body, 0)
        plsc.subcore_barrier()
        pltpu.sync_copy(acc.at[pl.ds(s * RZ, RZ)],
                        out_hbm.at[c, pl.ds(s * RZ, RZ)])

    @functools.partial(
        pl.kernel,
        out_type=jax.ShapeDtypeStruct((NC, ACC_N, D), jnp.float32),
        mesh=mesh,
        scratch_types=[
            pltpu.VMEM((HBMAX, KB), jnp.int32),
            pltpu.VMEM((HBMAX, KB), jnp.int32),
            [pltpu.VMEM((KB, D), jnp.float32)] * NBUF,
            pltpu.VMEM_SHARED((ACC_N, D), jnp.float32),
            [pltpu.SemaphoreType.DMA] * NBUF,
            [pltpu.SemaphoreType.DMA] * NBUF,
        ],
    )
    def sc_spmm(p_hbm, src_hbm, dst_hbm, zeros_hbm, out_hbm,
                srcb, dstb, rows, acc, gsem, ssem):
        c = lax.axis_index("c")
        s = lax.axis_index("s")
        pltpu.sync_copy(zeros_hbm.at[pl.ds(s * RZ, RZ)],
                        acc.at[pl.ds(s * RZ, RZ)])
        plsc.subcore_barrier()

        def run_edges(first_blk, nblk_c):
            hb = nblk_c // NPH
            for ph in range(NPH):
                base = first_blk + ph * hb
                pltpu.sync_copy(src_hbm.at[pl.ds(base, hb)],
                                srcb.at[pl.ds(0, hb)])
                pltpu.sync_copy(dst_hbm.at[pl.ds(base, hb)],
                                dstb.at[pl.ds(0, hb)])
                pltpu.async_copy(p_hbm.at[srcb.at[0]], rows[0], gsem[0])

                def body(i, carry):
                    for j in range(NBUF):
                        g = i * NBUF + j
                        bn = 1 - j
                        pltpu.make_async_copy(
                            p_hbm.at[srcb.at[g]], rows[j], gsem[j]).wait()
                        pltpu.async_copy(rows[j], acc.at[dstb.at[g]],
                                         ssem[j], add=True)
                        if j == 0:
                            @pl.when(i > 0)
                            def _():
                                pltpu.make_async_copy(
                                    rows[bn], acc.at[dstb.at[0]],
                                    ssem[bn]).wait()

                            pltpu.async_copy(
                                p_hbm.at[srcb.at[g + 1]], rows[bn], gsem[bn])
                        else:
                            @pl.when(i < hb // NBUF - 1)
                            def _():
                                pltpu.make_async_copy(
                                    rows[bn], acc.at[dstb.at[0]],
                                    ssem[bn]).wait()
                                pltpu.async_copy(
                                    p_hbm.at[srcb.at[g + 1]], rows[bn],
                                    gsem[bn])

                    return carry

                lax.fori_loop(0, hb // NBUF, body, 0)
                for b in range(NBUF):
                    pltpu.make_async_copy(
                        rows[b], acc.at[dstb.at[0]], ssem[b]).wait()

        @pl.when(c == 0)
        def _():
            run_edges(s * NBLK0, NBLK0)

        @pl.when(c == 1)
        def _():
            run_edges(NS * NBLK0 + s * NBLK1, NBLK1)

        plsc.subcore_barrier()
        pltpu.sync_copy(acc.at[pl.ds(s * RZ, RZ)],
                        out_hbm.at[c, pl.ds(s * RZ, RZ)])

    return sc_degree, sc_spmm


def _tc_entry(degA, degB, x, W):
    def body(dA, dB, x_ref, W_ref, out_ref):
        dis = lax.rsqrt(dA[...] + dB[...] + 1.0)
        out_ref[...] = jnp.dot(x_ref[...], W_ref[...],
                               preferred_element_type=jnp.float32) * dis

    return pl.pallas_call(
        body,
        grid=(NCHUNK,),
        in_specs=[
            pl.BlockSpec((BLK, 1), lambda i: (i, 0)),
            pl.BlockSpec((BLK, 1), lambda i: (i, 0)),
            pl.BlockSpec((BLK, D), lambda i: (i, 0)),
            pl.BlockSpec((D, D), lambda i: (0, 0)),
        ],
        out_specs=pl.BlockSpec((BLK, D), lambda i: (i, 0)),
        out_shape=jax.ShapeDtypeStruct((N, D), jnp.float32),
    )(degA, degB, x, W)


def _tc_mid(degA, degB, sA, sB, p, b, W):
    def body(dA, dB, sA_ref, sB_ref, p_ref, b_ref, W_ref, out_ref):
        dis = lax.rsqrt(dA[...] + dB[...] + 1.0)
        h = jnp.maximum(
            dis * (sA_ref[...] + sB_ref[...] + p_ref[...]) + b_ref[...], 0.0)
        out_ref[...] = jnp.dot(h, W_ref[...],
                               preferred_element_type=jnp.float32) * dis

    return pl.pallas_call(
        body,
        grid=(NCHUNK,),
        in_specs=[
            pl.BlockSpec((BLK, 1), lambda i: (i, 0)),
            pl.BlockSpec((BLK, 1), lambda i: (i, 0)),
            pl.BlockSpec((BLK, D), lambda i: (i, 0)),
            pl.BlockSpec((BLK, D), lambda i: (i, 0)),
            pl.BlockSpec((BLK, D), lambda i: (i, 0)),
            pl.BlockSpec((1, D), lambda i: (0, 0)),
            pl.BlockSpec((D, D), lambda i: (0, 0)),
        ],
        out_specs=pl.BlockSpec((BLK, D), lambda i: (i, 0)),
        out_shape=jax.ShapeDtypeStruct((N, D), jnp.float32),
    )(degA, degB, sA, sB, p, b, W)


def _tc_final(degA, degB, sA, sB, p, b, batch3, fW1, fb1, fW2, fb2):
    def body(dA, dB, sA_ref, sB_ref, p_ref, b_ref, bat_ref,
             fW1_ref, fb1_ref, fW2_ref, fb2_ref, out_ref, pool_acc, cnt_acc):
        i = pl.program_id(0)

        @pl.when(i == 0)
        def _():
            pool_acc[...] = jnp.zeros_like(pool_acc)
            cnt_acc[...] = jnp.zeros_like(cnt_acc)

        dis = lax.rsqrt(dA[...] + dB[...] + 1.0)
        h = jnp.maximum(
            dis * (sA_ref[...] + sB_ref[...] + p_ref[...]) + b_ref[...], 0.0)
        bb = bat_ref[0, 0, :]
        oh = (bb[None, :] == lax.broadcasted_iota(jnp.int32, (G, BLK), 0)
              ).astype(jnp.float32)
        pool_acc[...] += jnp.dot(oh, h, preferred_element_type=jnp.float32)
        cnt_acc[...] += jnp.sum(oh, axis=1, keepdims=True)

        @pl.when(i == NCHUNK - 1)
        def _():
            pooled = pool_acc[...] / jnp.maximum(cnt_acc[...], 1.0)
            z = jnp.maximum(
                jnp.dot(pooled, fW1_ref[...],
                        preferred_element_type=jnp.float32) + fb1_ref[...], 0.0)
            out_ref[...] = jnp.dot(z, fW2_ref[...],
                                   preferred_element_type=jnp.float32) + fb2_ref[...]

    return pl.pallas_call(
        body,
        grid=(NCHUNK,),
        in_specs=[
            pl.BlockSpec((BLK, 1), lambda i: (i, 0)),
            pl.BlockSpec((BLK, 1), lambda i: (i, 0)),
            pl.BlockSpec((BLK, D), lambda i: (i, 0)),
            pl.BlockSpec((BLK, D), lambda i: (i, 0)),
            pl.BlockSpec((BLK, D), lambda i: (i, 0)),
            pl.BlockSpec((1, D), lambda i: (0, 0)),
            pl.BlockSpec((1, 1, BLK), lambda i: (i, 0, 0)),
            pl.BlockSpec((D, G), lambda i: (0, 0)),
            pl.BlockSpec((1, G), lambda i: (0, 0)),
            pl.BlockSpec((G, 1), lambda i: (0, 0)),
            pl.BlockSpec((1, 1), lambda i: (0, 0)),
        ],
        out_specs=pl.BlockSpec((G, 1), lambda i: (0, 0)),
        out_shape=jax.ShapeDtypeStruct((G, 1), jnp.float32),
        scratch_shapes=[
            pltpu.VMEM((G, D), jnp.float32),
            pltpu.VMEM((G, 1), jnp.float32),
        ],
    )(degA, degB, sA, sB, p, b, batch3, fW1, fb1, fW2, fb2)


def kernel(x, edge_index, batch, W1, b1, W2, b2, W3, b3, fW1, fb1, fW2, fb2):
    src = edge_index[0]
    dst = edge_index[1]
    pad = EPAD - E
    src2 = jnp.concatenate([src, jnp.zeros((pad,), jnp.int32)]).reshape(-1, KB)
    dst2 = jnp.concatenate([dst, jnp.full((pad,), N, jnp.int32)]).reshape(-1, KB)

    ones_col = jnp.ones((KB, DEGW), jnp.float32)
    zeros_col = jnp.zeros((ACC_N, DEGW), jnp.float32)
    zeros_tab = jnp.zeros((ACC_N, D), jnp.float32)

    sc_degree, sc_spmm = _sc_kernels()
    deg = sc_degree(dst2, ones_col, zeros_col)
    degA, degB = deg[0, :, 0:1], deg[1, :, 0:1]

    b1r = b1.reshape(1, D)
    b2r = b2.reshape(1, D)
    b3r = b3.reshape(1, D)
    batch3 = batch.reshape(NCHUNK, 1, BLK)

    p1 = _tc_entry(degA, degB, x, W1)
    s1 = sc_spmm(p1, src2, dst2, zeros_tab)
    p2 = _tc_mid(degA, degB, s1[0], s1[1], p1, b1r, W2)
    s2 = sc_spmm(p2, src2, dst2, zeros_tab)
    p3 = _tc_mid(degA, degB, s2[0], s2[1], p2, b2r, W3)
    s3 = sc_spmm(p3, src2, dst2, zeros_tab)
    out = _tc_final(degA, degB, s3[0], s3[1], p3, b3r, batch3,
                    fW1, fb1.reshape(1, G), fW2, fb2.reshape(1, 1))
    return out.reshape(-1)
